# Initial kernel scaffold; baseline (speedup 1.0000x reference)
import jax
import jax.numpy as jnp
from jax.experimental import pallas as pl


def kernel(x, edge_index, W1, att_src1, att_dst1, b1, W2, att_src2, att_dst2, b2, Wc, bc):
    raise NotImplementedError("write your pallas kernel here")


# speedup vs baseline: 16.5395x; 16.5395x over previous
import functools

import jax
import jax.numpy as jnp
from jax import lax
from jax.experimental import pallas as pl
from jax.experimental.pallas import tpu as pltpu
from jax.experimental.pallas import tpu_sc as plsc

N = 50000
F_IN = 6
HID = 64
NUM_CLASSES = 2

NP = 50176
ROWS_PER_TILE = NP // 16
E_RAW = 800000
E_LOOP = E_RAW + N
EP = 851968
E_PER_TILE = EP // 16
SG_PER_TILE = E_PER_TILE // 1024

NB = 512
GRID = NP // NB


def _k1_body(x_ref, w1_ref, ms_ref, md_ref, h_ref, as_ref, ad_ref, cs_ref, cd_ref):
    i = pl.program_id(0)
    h = jnp.dot(x_ref[...], w1_ref[...], preferred_element_type=jnp.float32)
    a_s = jnp.dot(h, ms_ref[...], preferred_element_type=jnp.float32)
    a_d = jnp.dot(h, md_ref[...], preferred_element_type=jnp.float32)
    h_ref[...] = h
    as_ref[...] = a_s
    ad_ref[...] = a_d
    ms_blk = jnp.max(a_s[:, 0:4])
    md_blk = jnp.max(a_d[:, 0:4])

    @pl.when(i == 0)
    def _():
        cs_ref[...] = jnp.full((1, 16), -1e30, jnp.float32)
        cd_ref[...] = jnp.full((1, 16), -1e30, jnp.float32)

    cs_ref[...] = jnp.maximum(cs_ref[...], ms_blk)
    cd_ref[...] = jnp.maximum(cd_ref[...], md_blk)


def _dense1(x8, w1p, ms1, md1):
    return pl.pallas_call(
        _k1_body,
        grid=(GRID,),
        in_specs=[
            pl.BlockSpec((NB, 8), lambda i: (i, 0)),
            pl.BlockSpec((8, 4 * HID), lambda i: (0, 0)),
            pl.BlockSpec((4 * HID, 8), lambda i: (0, 0)),
            pl.BlockSpec((4 * HID, 8), lambda i: (0, 0)),
        ],
        out_specs=[
            pl.BlockSpec((NB, 4 * HID), lambda i: (i, 0)),
            pl.BlockSpec((NB, 8), lambda i: (i, 0)),
            pl.BlockSpec((NB, 8), lambda i: (i, 0)),
            pl.BlockSpec((1, 16), lambda i: (0, 0)),
            pl.BlockSpec((1, 16), lambda i: (0, 0)),
        ],
        out_shape=[
            jax.ShapeDtypeStruct((NP, 4 * HID), jnp.float32),
            jax.ShapeDtypeStruct((NP, 8), jnp.float32),
            jax.ShapeDtypeStruct((NP, 8), jnp.float32),
            jax.ShapeDtypeStruct((1, 16), jnp.float32),
            jax.ShapeDtypeStruct((1, 16), jnp.float32),
        ],
    )(x8, w1p, ms1, md1)


def _k3_body(msg_ref, d_ref, r1_ref, b1_ref, w2_ref, ms_ref, md_ref,
             h2_ref, as_ref, ad_ref, cs_ref, cd_ref):
    i = pl.program_id(0)
    d4 = d_ref[...][:, 0:4] + 1e-16
    dexp = jnp.dot(d4, r1_ref[...], preferred_element_type=jnp.float32)
    out1 = jnp.maximum(msg_ref[...] / dexp + b1_ref[...], 0.0)
    h2 = jnp.dot(out1, w2_ref[...], preferred_element_type=jnp.float32)
    a_s = jnp.dot(h2, ms_ref[...], preferred_element_type=jnp.float32)
    a_d = jnp.dot(h2, md_ref[...], preferred_element_type=jnp.float32)
    h2_ref[...] = h2
    as_ref[...] = a_s
    ad_ref[...] = a_d
    ms_blk = jnp.max(a_s[:, 0:1])
    md_blk = jnp.max(a_d[:, 0:1])

    @pl.when(i == 0)
    def _():
        cs_ref[...] = jnp.full((1, 16), -1e30, jnp.float32)
        cd_ref[...] = jnp.full((1, 16), -1e30, jnp.float32)

    cs_ref[...] = jnp.maximum(cs_ref[...], ms_blk)
    cd_ref[...] = jnp.maximum(cd_ref[...], md_blk)


def _dense2(msg1, d1, r1, b1r, w2, ms2, md2):
    return pl.pallas_call(
        _k3_body,
        grid=(GRID,),
        in_specs=[
            pl.BlockSpec((NB, 4 * HID), lambda i: (i, 0)),
            pl.BlockSpec((NB, 8), lambda i: (i, 0)),
            pl.BlockSpec((4, 4 * HID), lambda i: (0, 0)),
            pl.BlockSpec((1, 4 * HID), lambda i: (0, 0)),
            pl.BlockSpec((4 * HID, HID), lambda i: (0, 0)),
            pl.BlockSpec((HID, 8), lambda i: (0, 0)),
            pl.BlockSpec((HID, 8), lambda i: (0, 0)),
        ],
        out_specs=[
            pl.BlockSpec((NB, HID), lambda i: (i, 0)),
            pl.BlockSpec((NB, 8), lambda i: (i, 0)),
            pl.BlockSpec((NB, 8), lambda i: (i, 0)),
            pl.BlockSpec((1, 16), lambda i: (0, 0)),
            pl.BlockSpec((1, 16), lambda i: (0, 0)),
        ],
        out_shape=[
            jax.ShapeDtypeStruct((NP, HID), jnp.float32),
            jax.ShapeDtypeStruct((NP, 8), jnp.float32),
            jax.ShapeDtypeStruct((NP, 8), jnp.float32),
            jax.ShapeDtypeStruct((1, 16), jnp.float32),
            jax.ShapeDtypeStruct((1, 16), jnp.float32),
        ],
    )(msg1, d1, r1, b1r, w2, ms2, md2)


def _k6_body(msg_ref, d_ref, b2_ref, wc_ref, bc_ref, out_ref):
    d = d_ref[...][:, 0:1] + 1e-16
    out2 = jnp.maximum(msg_ref[...] / d + b2_ref[...], 0.0)
    lg = jnp.dot(out2, wc_ref[...], preferred_element_type=jnp.float32) + bc_ref[...]
    l0 = lg[:, 0:1]
    l1 = lg[:, 1:2]
    m = jnp.maximum(l0, l1)
    s = m + jnp.log(jnp.exp(l0 - m) + jnp.exp(l1 - m))
    out_ref[...] = lg[:, 0:2] - s


def _dense3(msg2, d2, b2r, wcp, bcp):
    return pl.pallas_call(
        _k6_body,
        grid=(GRID,),
        in_specs=[
            pl.BlockSpec((NB, HID), lambda i: (i, 0)),
            pl.BlockSpec((NB, 8), lambda i: (i, 0)),
            pl.BlockSpec((1, HID), lambda i: (0, 0)),
            pl.BlockSpec((HID, 128), lambda i: (0, 0)),
            pl.BlockSpec((1, 128), lambda i: (0, 0)),
        ],
        out_specs=pl.BlockSpec((NB, 2), lambda i: (i, 0)),
        out_shape=jax.ShapeDtypeStruct((NP, 2), jnp.float32),
    )(msg2, d2, b2r, wcp, bcp)


def _make_sc_layer(heads):
    chunks = 2 * heads
    cps = heads

    mesh = plsc.VectorSubcoreMesh(core_axis_name="c", subcore_axis_name="s")

    @functools.partial(
        pl.kernel,
        out_type=[
            jax.ShapeDtypeStruct((chunks * NP, 32), jnp.float32),
            jax.ShapeDtypeStruct((heads * NP,), jnp.float32),
        ],
        mesh=mesh,
        compiler_params=pltpu.CompilerParams(use_tc_tiling_on_sc=False),
        scratch_types=[
            pltpu.VMEM((1024,), jnp.int32),
            pltpu.VMEM((8, 128), jnp.int32),
            pltpu.VMEM((128,), jnp.int32),
            pltpu.VMEM((128,), jnp.int32),
            pltpu.VMEM((128,), jnp.int32),
            pltpu.VMEM((128,), jnp.float32),
            pltpu.VMEM((128,), jnp.float32),
            pltpu.VMEM((128,), jnp.float32),
            pltpu.VMEM((128, 32), jnp.float32),
            pltpu.VMEM((16,), jnp.float32),
            pltpu.VMEM((16,), jnp.float32),
            pltpu.VMEM((ROWS_PER_TILE,), jnp.float32),
            pltpu.VMEM((ROWS_PER_TILE,), jnp.float32),
            pltpu.VMEM_SHARED((NP, 32), jnp.float32),
            pltpu.VMEM_SHARED((NP,), jnp.float32),
        ],
    )
    def sc_layer(src_hbm, dst2_hbm, asf_hbm, adf_hbm, hf_hbm,
                 zeros32_hbm, csv_hbm, cdv_hbm,
                 msg_out, den_out,
                 srcb, dstb2, asidx, adidx, hidx, asb, adb, fstage,
                 rows, csb, cdb, zbuf, wbuf, acc, dsh):
        scid = lax.axis_index("c")
        sid = lax.axis_index("s")
        row0 = sid * ROWS_PER_TILE
        tile_e0 = sid * E_PER_TILE

        for k in range(ROWS_PER_TILE // 16):
            zbuf[pl.ds(k * 16, 16)] = jnp.zeros((16,), jnp.float32)
        pltpu.sync_copy(csv_hbm, csb)
        pltpu.sync_copy(cdv_hbm, cdb)
        cvec = csb[...] + cdb[...]
        cval = cvec[0]

        for c in range(cps):
            cg = scid * cps + c
            head = cg // 2
            primary = (cg % 2) == 0
            hoff = head * NP
            goff = cg * NP

            pltpu.sync_copy(zeros32_hbm, acc.at[pl.ds(row0, ROWS_PER_TILE)])

            @pl.when(primary)
            def _():
                pltpu.sync_copy(zbuf, dsh.at[pl.ds(row0, ROWS_PER_TILE)])

            plsc.subcore_barrier()

            def sg_body(sg, carry):
                eoff = tile_e0 + sg * 1024
                pltpu.sync_copy(src_hbm.at[pl.ds(eoff, 1024)], srcb)
                pltpu.sync_copy(
                    dst2_hbm.at[pl.ds(pl.multiple_of(eoff // 128, 8), 8)], dstb2)

                def gg_body(gg, carry2):
                    g0 = gg * 128
                    for k in range(8):
                        sl = pl.ds(k * 16, 16)
                        src16 = srcb[pl.ds(g0 + k * 16, 16)]
                        dst16 = dstb2[gg, pl.ds(k * 16, 16)]
                        asidx[sl] = src16 + hoff
                        adidx[sl] = dst16 + hoff
                        hidx[sl] = src16 + goff
                    pltpu.sync_copy(asf_hbm.at[asidx], asb)
                    pltpu.sync_copy(adf_hbm.at[adidx], adb)
                    pltpu.sync_copy(hf_hbm.at[hidx], rows)
                    for k in range(8):
                        sl = pl.ds(k * 16, 16)
                        z = asb[sl] + adb[sl]
                        t = jnp.maximum(z, 0.2 * z) - cval
                        fstage[sl] = jnp.exp(t)

                    @pl.when(primary)
                    def _():
                        pltpu.sync_copy(fstage, dsh.at[dstb2.at[gg]], add=True)

                    def mul_body(kk, carry3):
                        f16 = fstage[pl.ds(kk * 16, 16)]
                        for j in range(16):
                            e = kk * 16 + j
                            fs = f16[j]
                            rows[e, pl.ds(0, 16)] = rows[e, pl.ds(0, 16)] * fs
                            rows[e, pl.ds(16, 16)] = rows[e, pl.ds(16, 16)] * fs
                        return carry3

                    lax.fori_loop(0, 8, mul_body, 0)
                    pltpu.sync_copy(rows, acc.at[dstb2.at[gg]], add=True)
                    return carry2

                lax.fori_loop(0, 8, gg_body, 0)
                return carry

            lax.fori_loop(0, SG_PER_TILE, sg_body, 0)
            plsc.subcore_barrier()

            pltpu.sync_copy(
                acc.at[pl.ds(row0, ROWS_PER_TILE)],
                msg_out.at[pl.ds(pl.multiple_of(goff + row0, 8),
                                 ROWS_PER_TILE)])

            @pl.when(primary)
            def _():
                pltpu.sync_copy(dsh.at[pl.ds(row0, ROWS_PER_TILE)], wbuf)
                pltpu.sync_copy(
                    wbuf,
                    den_out.at[pl.ds(pl.multiple_of(hoff + row0, 8),
                                     ROWS_PER_TILE)])

            plsc.subcore_barrier()

    return sc_layer


_sc_layer1 = _make_sc_layer(4)
_sc_layer2 = _make_sc_layer(1)


def _att_matrix(att, heads, dim):
    m = jnp.zeros((heads * dim, 8), jnp.float32)
    for h in range(heads):
        m = m.at[h * dim:(h + 1) * dim, h].set(att[0, h, :])
    return m


def kernel(x, edge_index, W1, att_src1, att_dst1, b1, W2, att_src2, att_dst2,
           b2, Wc, bc):
    f32 = jnp.float32

    x8 = jnp.zeros((NP, 8), f32).at[:N, :F_IN].set(x)
    w1p = jnp.zeros((8, 4 * HID), f32).at[:F_IN].set(W1)
    ms1 = _att_matrix(att_src1, 4, HID)
    md1 = _att_matrix(att_dst1, 4, HID)
    ms2 = _att_matrix(att_src2, 1, HID)
    md2 = _att_matrix(att_dst2, 1, HID)

    loops = jnp.arange(N, dtype=jnp.int32)
    pad = jnp.full((EP - E_LOOP,), NP - 1, jnp.int32)
    src = jnp.concatenate([edge_index[0].astype(jnp.int32), loops, pad])
    dst = jnp.concatenate([edge_index[1].astype(jnp.int32), loops, pad])
    dst2 = dst.reshape(EP // 128, 128)

    zeros32 = jnp.zeros((ROWS_PER_TILE, 32), f32)

    h1, as1, ad1, cs1, cd1 = _dense1(x8, w1p, ms1, md1)
    asf1 = as1[:, 0:4].T.reshape(4 * NP)
    adf1 = ad1[:, 0:4].T.reshape(4 * NP)
    hf1 = h1.reshape(NP, 8, 32).transpose(1, 0, 2).reshape(8 * NP, 32)
    msg1, den1 = _sc_layer1(src, dst2, asf1, adf1, hf1,
                            zeros32, cs1.reshape(16), cd1.reshape(16))
    msg1r = msg1.reshape(8, NP, 32).transpose(1, 0, 2).reshape(NP, 4 * HID)
    d1 = jnp.zeros((NP, 8), f32).at[:, 0:4].set(den1.reshape(4, NP).T)

    b1r = b1.reshape(1, 4 * HID)
    r1 = jnp.repeat(jnp.eye(4, dtype=f32), HID, axis=1)
    h2, as2, ad2, cs2, cd2 = _dense2(msg1r, d1, r1, b1r, W2, ms2, md2)
    asf2 = as2[:, 0]
    adf2 = ad2[:, 0]
    hf2 = h2.reshape(NP, 2, 32).transpose(1, 0, 2).reshape(2 * NP, 32)
    msg2, den2 = _sc_layer2(src, dst2, asf2, adf2, hf2,
                            zeros32, cs2.reshape(16), cd2.reshape(16))
    msg2r = msg2.reshape(2, NP, 32).transpose(1, 0, 2).reshape(NP, HID)
    d2 = jnp.zeros((NP, 8), f32).at[:, 0:1].set(den2.reshape(1, NP).T)

    b2r = b2.reshape(1, HID)
    wcp = jnp.zeros((HID, 128), f32).at[:, 0:NUM_CLASSES].set(Wc)
    bcp = jnp.zeros((1, 128), f32).at[0, 0:NUM_CLASSES].set(bc)
    out = _dense3(msg2r, d2, b2r, wcp, bcp)
    return out[:N]


# speedup vs baseline: 30.8721x; 1.8666x over previous
import functools

import jax
import jax.numpy as jnp
from jax import lax
from jax.experimental import pallas as pl
from jax.experimental.pallas import tpu as pltpu
from jax.experimental.pallas import tpu_sc as plsc

N = 50000
F_IN = 6
HID = 64
NUM_CLASSES = 2

NP = 50176
ROWS_PER_TILE = NP // 16
E_RAW = 800000
E_LOOP = E_RAW + N
EP = 851968
E_PER_TILE = EP // 16
SG_PER_TILE = E_PER_TILE // 1024

NB = 512
GRID = NP // NB


def _k1_body(x_ref, w1_ref, ms_ref, md_ref, h_ref, as_ref, ad_ref, cs_ref, cd_ref):
    i = pl.program_id(0)
    h = jnp.dot(x_ref[...], w1_ref[...], preferred_element_type=jnp.float32)
    a_s = jnp.dot(h, ms_ref[...], preferred_element_type=jnp.float32)
    a_d = jnp.dot(h, md_ref[...], preferred_element_type=jnp.float32)
    h_ref[...] = h
    as_ref[...] = a_s
    ad_ref[...] = a_d
    ms_blk = jnp.max(a_s[:, 0:4])
    md_blk = jnp.max(a_d[:, 0:4])

    @pl.when(i == 0)
    def _():
        cs_ref[...] = jnp.full((1, 16), -1e30, jnp.float32)
        cd_ref[...] = jnp.full((1, 16), -1e30, jnp.float32)

    cs_ref[...] = jnp.maximum(cs_ref[...], ms_blk)
    cd_ref[...] = jnp.maximum(cd_ref[...], md_blk)


def _dense1(x8, w1p, ms1, md1):
    return pl.pallas_call(
        _k1_body,
        grid=(GRID,),
        in_specs=[
            pl.BlockSpec((NB, 8), lambda i: (i, 0)),
            pl.BlockSpec((8, 4 * HID), lambda i: (0, 0)),
            pl.BlockSpec((4 * HID, 8), lambda i: (0, 0)),
            pl.BlockSpec((4 * HID, 8), lambda i: (0, 0)),
        ],
        out_specs=[
            pl.BlockSpec((NB, 4 * HID), lambda i: (i, 0)),
            pl.BlockSpec((NB, 8), lambda i: (i, 0)),
            pl.BlockSpec((NB, 8), lambda i: (i, 0)),
            pl.BlockSpec((1, 16), lambda i: (0, 0)),
            pl.BlockSpec((1, 16), lambda i: (0, 0)),
        ],
        out_shape=[
            jax.ShapeDtypeStruct((NP, 4 * HID), jnp.float32),
            jax.ShapeDtypeStruct((NP, 8), jnp.float32),
            jax.ShapeDtypeStruct((NP, 8), jnp.float32),
            jax.ShapeDtypeStruct((1, 16), jnp.float32),
            jax.ShapeDtypeStruct((1, 16), jnp.float32),
        ],
    )(x8, w1p, ms1, md1)


def _k3_body(msg_ref, da_ref, db_ref, r1_ref, b1_ref, w2_ref, ms_ref, md_ref,
             h2_ref, as_ref, ad_ref, cs_ref, cd_ref):
    i = pl.program_id(0)
    d4 = (da_ref[...] + db_ref[...])[:, 0:4] + 1e-16
    dexp = jnp.dot(d4, r1_ref[...], preferred_element_type=jnp.float32)
    out1 = jnp.maximum(msg_ref[...] / dexp + b1_ref[...], 0.0)
    h2 = jnp.dot(out1, w2_ref[...], preferred_element_type=jnp.float32)
    a_s = jnp.dot(h2, ms_ref[...], preferred_element_type=jnp.float32)
    a_d = jnp.dot(h2, md_ref[...], preferred_element_type=jnp.float32)
    h2_ref[...] = h2
    as_ref[...] = a_s
    ad_ref[...] = a_d
    ms_blk = jnp.max(a_s[:, 0:1])
    md_blk = jnp.max(a_d[:, 0:1])

    @pl.when(i == 0)
    def _():
        cs_ref[...] = jnp.full((1, 16), -1e30, jnp.float32)
        cd_ref[...] = jnp.full((1, 16), -1e30, jnp.float32)

    cs_ref[...] = jnp.maximum(cs_ref[...], ms_blk)
    cd_ref[...] = jnp.maximum(cd_ref[...], md_blk)


def _dense2(msg1, da, db, r1, b1r, w2, ms2, md2):
    return pl.pallas_call(
        _k3_body,
        grid=(GRID,),
        in_specs=[
            pl.BlockSpec((NB, 4 * HID), lambda i: (i, 0)),
            pl.BlockSpec((NB, 8), lambda i: (i, 0)),
            pl.BlockSpec((NB, 8), lambda i: (i, 0)),
            pl.BlockSpec((4, 4 * HID), lambda i: (0, 0)),
            pl.BlockSpec((1, 4 * HID), lambda i: (0, 0)),
            pl.BlockSpec((4 * HID, HID), lambda i: (0, 0)),
            pl.BlockSpec((HID, 8), lambda i: (0, 0)),
            pl.BlockSpec((HID, 8), lambda i: (0, 0)),
        ],
        out_specs=[
            pl.BlockSpec((NB, HID), lambda i: (i, 0)),
            pl.BlockSpec((NB, 8), lambda i: (i, 0)),
            pl.BlockSpec((NB, 8), lambda i: (i, 0)),
            pl.BlockSpec((1, 16), lambda i: (0, 0)),
            pl.BlockSpec((1, 16), lambda i: (0, 0)),
        ],
        out_shape=[
            jax.ShapeDtypeStruct((NP, HID), jnp.float32),
            jax.ShapeDtypeStruct((NP, 8), jnp.float32),
            jax.ShapeDtypeStruct((NP, 8), jnp.float32),
            jax.ShapeDtypeStruct((1, 16), jnp.float32),
            jax.ShapeDtypeStruct((1, 16), jnp.float32),
        ],
    )(msg1, da, db, r1, b1r, w2, ms2, md2)


def _k6_body(msg_ref, da_ref, db_ref, b2_ref, wc_ref, bc_ref, out_ref):
    d = (da_ref[...] + db_ref[...])[:, 0:1] + 1e-16
    out2 = jnp.maximum(msg_ref[...] / d + b2_ref[...], 0.0)
    lg = jnp.dot(out2, wc_ref[...], preferred_element_type=jnp.float32) + bc_ref[...]
    l0 = lg[:, 0:1]
    l1 = lg[:, 1:2]
    m = jnp.maximum(l0, l1)
    s = m + jnp.log(jnp.exp(l0 - m) + jnp.exp(l1 - m))
    out_ref[...] = lg[:, 0:2] - s


def _dense3(msg2, da, db, b2r, wcp, bcp):
    return pl.pallas_call(
        _k6_body,
        grid=(GRID,),
        in_specs=[
            pl.BlockSpec((NB, HID), lambda i: (i, 0)),
            pl.BlockSpec((NB, 8), lambda i: (i, 0)),
            pl.BlockSpec((NB, 8), lambda i: (i, 0)),
            pl.BlockSpec((1, HID), lambda i: (0, 0)),
            pl.BlockSpec((HID, 128), lambda i: (0, 0)),
            pl.BlockSpec((1, 128), lambda i: (0, 0)),
        ],
        out_specs=pl.BlockSpec((NB, 2), lambda i: (i, 0)),
        out_shape=jax.ShapeDtypeStruct((NP, 2), jnp.float32),
    )(msg2, da, db, b2r, wcp, bcp)


SG_A = EP // 32 // 1024


def _make_pass_a(heads):
    mesh = plsc.VectorSubcoreMesh(core_axis_name="c", subcore_axis_name="s")

    @functools.partial(
        pl.kernel,
        out_type=[
            jax.ShapeDtypeStruct((heads * EP,), jnp.float32),
            jax.ShapeDtypeStruct((2 * NP, 8), jnp.float32),
        ],
        mesh=mesh,
        compiler_params=pltpu.CompilerParams(use_tc_tiling_on_sc=False, needs_layout_passes=False),
        scratch_types=[
            pltpu.VMEM((1024,), jnp.int32),
            pltpu.VMEM((8, 128), jnp.int32),
            pltpu.VMEM((128, 8), jnp.float32),
            pltpu.VMEM((128, 8), jnp.float32),
            pltpu.VMEM((128, 8), jnp.float32),
            pltpu.VMEM((8 * 1024,), jnp.float32),
            pltpu.VMEM((16,), jnp.int32),
            pltpu.VMEM((16,), jnp.int32),
            pltpu.VMEM((16,), jnp.int32),
            pltpu.VMEM((16,), jnp.float32),
            pltpu.VMEM((16,), jnp.float32),
            pltpu.VMEM_SHARED((NP, 8), jnp.float32),
        ],
    )
    def pass_a(src_hbm, dst2_hbm, as_hbm, ad_hbm, zeros8_hbm, csv_hbm, cdv_hbm,
               f_out, den_out,
               srcb, dstb2, asrow, adrow, fbuf, ftmp, patb, rowb, colb,
               csb, cdb, dsh8):
        scid = lax.axis_index("c")
        sid = lax.axis_index("s")
        row0 = sid * ROWS_PER_TILE
        tile_e0 = (scid * 16 + sid) * (EP // 32)

        pltpu.sync_copy(csv_hbm, csb)
        pltpu.sync_copy(cdv_hbm, cdb)
        cvec = csb[...] + cdb[...]
        cval = cvec[0]
        lanes = lax.iota(jnp.int32, 16)
        patb[...] = (lanes & 7) * 1024 + (lanes >> 3)
        rowb[...] = lanes >> 3
        colb[...] = lanes & 7

        pltpu.sync_copy(zeros8_hbm, dsh8.at[pl.ds(row0, ROWS_PER_TILE)])
        plsc.subcore_barrier()

        def sg_body(sg, carry):
            eoff = tile_e0 + sg * 1024
            pltpu.sync_copy(src_hbm.at[pl.ds(eoff, 1024)], srcb)
            pltpu.sync_copy(
                dst2_hbm.at[pl.ds(pl.multiple_of(eoff // 128, 8), 8)], dstb2)

            def gg_body(gg, carry2):
                pltpu.sync_copy(as_hbm.at[srcb.at[pl.ds(gg * 128, 128)]], asrow)
                pltpu.sync_copy(ad_hbm.at[dstb2.at[gg]], adrow)

                pat0 = patb[...] + gg * 128

                def kk_body(kk, carry3):
                    ri = rowb[...] + 2 * kk
                    ci = colb[...]
                    z = (plsc.load_gather(asrow, [ri, ci])
                         + plsc.load_gather(adrow, [ri, ci]))
                    f16 = jnp.exp(jnp.maximum(z, 0.2 * z) - cval)
                    plsc.store_scatter(fbuf, [ri, ci], f16)
                    plsc.store_scatter(ftmp, [pat0 + 2 * kk], f16)
                    return carry3

                lax.fori_loop(0, 64, kk_body, 0)
                pltpu.sync_copy(fbuf, dsh8.at[dstb2.at[gg]], add=True)
                return carry2

            lax.fori_loop(0, 8, gg_body, 0)
            for h in range(heads):
                pltpu.sync_copy(
                    ftmp.at[pl.ds(h * 1024, 1024)],
                    f_out.at[pl.ds(pl.multiple_of(h * EP + eoff, 8), 1024)])
            return carry

        lax.fori_loop(0, SG_A, sg_body, 0)
        plsc.subcore_barrier()
        pltpu.sync_copy(
            dsh8.at[pl.ds(row0, ROWS_PER_TILE)],
            den_out.at[pl.ds(pl.multiple_of(scid * NP + row0, 8),
                             ROWS_PER_TILE)])

    return pass_a


def _make_sc_layer(heads):
    chunks = 2 * heads
    cps = heads

    mesh = plsc.VectorSubcoreMesh(core_axis_name="c", subcore_axis_name="s")

    @functools.partial(
        pl.kernel,
        out_type=jax.ShapeDtypeStruct((chunks * NP, 32), jnp.float32),
        mesh=mesh,
        compiler_params=pltpu.CompilerParams(use_tc_tiling_on_sc=False, needs_layout_passes=False),
        scratch_types=[
            pltpu.VMEM((1024,), jnp.int32),
            pltpu.VMEM((1024,), jnp.int32),
            pltpu.VMEM((1024,), jnp.float32),
            pltpu.VMEM((512,), jnp.int32),
            pltpu.VMEM((512, 32), jnp.float32),
            pltpu.VMEM_SHARED((NP, 32), jnp.float32),
        ],
    )
    def sc_layer(src_hbm, dst_hbm, f_hbm, hf_hbm, zeros32_hbm,
                 msg_out,
                 srcb, dstb, fsl, hidx, rows, acc):
        scid = lax.axis_index("c")
        sid = lax.axis_index("s")
        row0 = sid * ROWS_PER_TILE
        tile_e0 = sid * E_PER_TILE

        for c in range(cps):
            cg = scid * cps + c
            head = cg // 2
            goff = cg * NP

            pltpu.sync_copy(zeros32_hbm, acc.at[pl.ds(row0, ROWS_PER_TILE)])
            plsc.subcore_barrier()

            def sg_body(sg, carry):
                eoff = tile_e0 + sg * 1024
                pltpu.sync_copy(src_hbm.at[pl.ds(eoff, 1024)], srcb)
                pltpu.sync_copy(dst_hbm.at[pl.ds(eoff, 1024)], dstb)
                pltpu.sync_copy(
                    f_hbm.at[pl.ds(pl.multiple_of(head * EP + eoff, 8), 1024)],
                    fsl)

                for hh in range(2):
                    h0 = hh * 512

                    def idx_body(k, carry2):
                        hidx[pl.ds(k * 16, 16)] = (
                            srcb[pl.ds(h0 + k * 16, 16)] + goff)
                        return carry2

                    lax.fori_loop(0, 32, idx_body, 0)
                    pltpu.sync_copy(hf_hbm.at[hidx], rows)

                    def mul_body(kk, carry3):
                        f16 = fsl[pl.ds(h0 + kk * 16, 16)]
                        for j in range(16):
                            e = kk * 16 + j
                            fs = f16[j]
                            rows[e, pl.ds(0, 16)] = rows[e, pl.ds(0, 16)] * fs
                            rows[e, pl.ds(16, 16)] = (
                                rows[e, pl.ds(16, 16)] * fs)
                        return carry3

                    lax.fori_loop(0, 32, mul_body, 0)
                    pltpu.sync_copy(rows, acc.at[dstb.at[pl.ds(h0, 512)]],
                                    add=True)
                return carry

            lax.fori_loop(0, SG_PER_TILE, sg_body, 0)
            plsc.subcore_barrier()

            pltpu.sync_copy(
                acc.at[pl.ds(row0, ROWS_PER_TILE)],
                msg_out.at[pl.ds(pl.multiple_of(goff + row0, 8),
                                 ROWS_PER_TILE)])
            plsc.subcore_barrier()

    return sc_layer


_pass_a1 = _make_pass_a(4)
_pass_a2 = _make_pass_a(1)
_sc_layer1 = _make_sc_layer(4)
_sc_layer2 = _make_sc_layer(1)


def _att_matrix(att, heads, dim):
    m = jnp.zeros((heads * dim, 8), jnp.float32)
    for h in range(heads):
        m = m.at[h * dim:(h + 1) * dim, h].set(att[0, h, :])
    return m


def kernel(x, edge_index, W1, att_src1, att_dst1, b1, W2, att_src2, att_dst2,
           b2, Wc, bc):
    f32 = jnp.float32

    x8 = jnp.zeros((NP, 8), f32).at[:N, :F_IN].set(x)
    w1p = jnp.zeros((8, 4 * HID), f32).at[:F_IN].set(W1)
    ms1 = _att_matrix(att_src1, 4, HID)
    md1 = _att_matrix(att_dst1, 4, HID)
    ms2 = _att_matrix(att_src2, 1, HID)
    md2 = _att_matrix(att_dst2, 1, HID)

    loops = jnp.arange(N, dtype=jnp.int32)
    pad = jnp.full((EP - E_LOOP,), NP - 1, jnp.int32)
    src = jnp.concatenate([edge_index[0].astype(jnp.int32), loops, pad])
    dst = jnp.concatenate([edge_index[1].astype(jnp.int32), loops, pad])
    dst2 = dst.reshape(EP // 128, 128)

    zeros32 = jnp.zeros((ROWS_PER_TILE, 32), f32)
    zeros8 = jnp.zeros((ROWS_PER_TILE, 8), f32)

    h1, as1, ad1, cs1, cd1 = _dense1(x8, w1p, ms1, md1)
    hf1 = h1.reshape(NP, 8, 32).transpose(1, 0, 2).reshape(8 * NP, 32)
    f1, den1 = _pass_a1(src, dst2, as1, ad1, zeros8,
                        cs1.reshape(16), cd1.reshape(16))
    msg1 = _sc_layer1(src, dst, f1, hf1, zeros32)
    msg1r = msg1.reshape(8, NP, 32).transpose(1, 0, 2).reshape(NP, 4 * HID)

    b1r = b1.reshape(1, 4 * HID)
    r1 = jnp.repeat(jnp.eye(4, dtype=f32), HID, axis=1)
    h2, as2, ad2, cs2, cd2 = _dense2(msg1r, den1[:NP], den1[NP:], r1, b1r,
                                     W2, ms2, md2)
    hf2 = h2.reshape(NP, 2, 32).transpose(1, 0, 2).reshape(2 * NP, 32)
    f2, den2 = _pass_a2(src, dst2, as2, ad2, zeros8,
                        cs2.reshape(16), cd2.reshape(16))
    msg2 = _sc_layer2(src, dst, f2, hf2, zeros32)
    msg2r = msg2.reshape(2, NP, 32).transpose(1, 0, 2).reshape(NP, HID)

    b2r = b2.reshape(1, HID)
    wcp = jnp.zeros((HID, 128), f32).at[:, 0:NUM_CLASSES].set(Wc)
    bcp = jnp.zeros((1, 128), f32).at[0, 0:NUM_CLASSES].set(bc)
    out = _dense3(msg2r, den2[:NP], den2[NP:], b2r, wcp, bcp)
    return out[:N]


# speedup vs baseline: 39.4637x; 1.2783x over previous
import functools

import jax
import jax.numpy as jnp
from jax import lax
from jax.experimental import pallas as pl
from jax.experimental.pallas import tpu as pltpu
from jax.experimental.pallas import tpu_sc as plsc

N = 50000
F_IN = 6
HID = 64
NUM_CLASSES = 2

NP = 50176
ROWS_PER_TILE = NP // 16
E_RAW = 800000
E_LOOP = E_RAW + N
EP = 851968
E_PER_TILE = EP // 16
SG_PER_TILE = E_PER_TILE // 1024

NB = 512
GRID = NP // NB


def _k1_body(x_ref, w1_ref, ms_ref, md_ref, h_ref, as_ref, ad_ref, cs_ref, cd_ref):
    i = pl.program_id(0)
    h = jnp.dot(x_ref[...], w1_ref[...], preferred_element_type=jnp.float32)
    a_s = jnp.dot(h, ms_ref[...], preferred_element_type=jnp.float32)
    a_d = jnp.dot(h, md_ref[...], preferred_element_type=jnp.float32)
    h_ref[...] = h
    as_ref[...] = a_s
    ad_ref[...] = a_d
    ms_blk = jnp.max(a_s[:, 0:4])
    md_blk = jnp.max(a_d[:, 0:4])

    @pl.when(i == 0)
    def _():
        cs_ref[...] = jnp.full((1, 16), -1e30, jnp.float32)
        cd_ref[...] = jnp.full((1, 16), -1e30, jnp.float32)

    cs_ref[...] = jnp.maximum(cs_ref[...], ms_blk)
    cd_ref[...] = jnp.maximum(cd_ref[...], md_blk)


def _dense1(x8, w1p, ms1, md1):
    return pl.pallas_call(
        _k1_body,
        grid=(GRID,),
        in_specs=[
            pl.BlockSpec((NB, 8), lambda i: (i, 0)),
            pl.BlockSpec((8, 4 * HID), lambda i: (0, 0)),
            pl.BlockSpec((4 * HID, 8), lambda i: (0, 0)),
            pl.BlockSpec((4 * HID, 8), lambda i: (0, 0)),
        ],
        out_specs=[
            pl.BlockSpec((NB, 4 * HID), lambda i: (i, 0)),
            pl.BlockSpec((NB, 8), lambda i: (i, 0)),
            pl.BlockSpec((NB, 8), lambda i: (i, 0)),
            pl.BlockSpec((1, 16), lambda i: (0, 0)),
            pl.BlockSpec((1, 16), lambda i: (0, 0)),
        ],
        out_shape=[
            jax.ShapeDtypeStruct((NP, 4 * HID), jnp.float32),
            jax.ShapeDtypeStruct((NP, 8), jnp.float32),
            jax.ShapeDtypeStruct((NP, 8), jnp.float32),
            jax.ShapeDtypeStruct((1, 16), jnp.float32),
            jax.ShapeDtypeStruct((1, 16), jnp.float32),
        ],
    )(x8, w1p, ms1, md1)


def _k3_body(msg_ref, da_ref, db_ref, r1_ref, b1_ref, w2_ref, ms_ref, md_ref,
             h2_ref, as_ref, ad_ref, cs_ref, cd_ref):
    i = pl.program_id(0)
    d4 = (da_ref[...] + db_ref[...])[:, 0:4] + 1e-16
    dexp = jnp.dot(d4, r1_ref[...], preferred_element_type=jnp.float32)
    out1 = jnp.maximum(msg_ref[...] / dexp + b1_ref[...], 0.0)
    h2 = jnp.dot(out1, w2_ref[...], preferred_element_type=jnp.float32)
    a_s = jnp.dot(h2, ms_ref[...], preferred_element_type=jnp.float32)
    a_d = jnp.dot(h2, md_ref[...], preferred_element_type=jnp.float32)
    h2_ref[...] = h2
    as_ref[...] = a_s
    ad_ref[...] = a_d
    ms_blk = jnp.max(a_s[:, 0:1])
    md_blk = jnp.max(a_d[:, 0:1])

    @pl.when(i == 0)
    def _():
        cs_ref[...] = jnp.full((1, 16), -1e30, jnp.float32)
        cd_ref[...] = jnp.full((1, 16), -1e30, jnp.float32)

    cs_ref[...] = jnp.maximum(cs_ref[...], ms_blk)
    cd_ref[...] = jnp.maximum(cd_ref[...], md_blk)


def _dense2(msg1, da, db, r1, b1r, w2, ms2, md2):
    return pl.pallas_call(
        _k3_body,
        grid=(GRID,),
        in_specs=[
            pl.BlockSpec((NB, 4 * HID), lambda i: (i, 0)),
            pl.BlockSpec((NB, 8), lambda i: (i, 0)),
            pl.BlockSpec((NB, 8), lambda i: (i, 0)),
            pl.BlockSpec((4, 4 * HID), lambda i: (0, 0)),
            pl.BlockSpec((1, 4 * HID), lambda i: (0, 0)),
            pl.BlockSpec((4 * HID, HID), lambda i: (0, 0)),
            pl.BlockSpec((HID, 8), lambda i: (0, 0)),
            pl.BlockSpec((HID, 8), lambda i: (0, 0)),
        ],
        out_specs=[
            pl.BlockSpec((NB, HID), lambda i: (i, 0)),
            pl.BlockSpec((NB, 8), lambda i: (i, 0)),
            pl.BlockSpec((NB, 8), lambda i: (i, 0)),
            pl.BlockSpec((1, 16), lambda i: (0, 0)),
            pl.BlockSpec((1, 16), lambda i: (0, 0)),
        ],
        out_shape=[
            jax.ShapeDtypeStruct((NP, HID), jnp.float32),
            jax.ShapeDtypeStruct((NP, 8), jnp.float32),
            jax.ShapeDtypeStruct((NP, 8), jnp.float32),
            jax.ShapeDtypeStruct((1, 16), jnp.float32),
            jax.ShapeDtypeStruct((1, 16), jnp.float32),
        ],
    )(msg1, da, db, r1, b1r, w2, ms2, md2)


def _k6_body(msg_ref, da_ref, db_ref, b2_ref, wc_ref, bc_ref, out_ref):
    d = (da_ref[...] + db_ref[...]) + 1e-16
    out2 = jnp.maximum(msg_ref[...] / d + b2_ref[...], 0.0)
    lg = jnp.dot(out2, wc_ref[...], preferred_element_type=jnp.float32) + bc_ref[...]
    l0 = lg[:, 0:1]
    l1 = lg[:, 1:2]
    m = jnp.maximum(l0, l1)
    s = m + jnp.log(jnp.exp(l0 - m) + jnp.exp(l1 - m))
    out_ref[...] = lg[:, 0:2] - s


def _dense3(msg2, da, db, b2r, wcp, bcp):
    return pl.pallas_call(
        _k6_body,
        grid=(GRID,),
        in_specs=[
            pl.BlockSpec((NB, HID), lambda i: (i, 0)),
            pl.BlockSpec((NB, 1), lambda i: (i, 0)),
            pl.BlockSpec((NB, 1), lambda i: (i, 0)),
            pl.BlockSpec((1, HID), lambda i: (0, 0)),
            pl.BlockSpec((HID, 128), lambda i: (0, 0)),
            pl.BlockSpec((1, 128), lambda i: (0, 0)),
        ],
        out_specs=pl.BlockSpec((NB, 2), lambda i: (i, 0)),
        out_shape=jax.ShapeDtypeStruct((NP, 2), jnp.float32),
    )(msg2, da, db, b2r, wcp, bcp)


SG_A = EP // 32 // 1024


def _make_pass_a4():
    mesh = plsc.VectorSubcoreMesh(core_axis_name="c", subcore_axis_name="s")

    @functools.partial(
        pl.kernel,
        out_type=[
            jax.ShapeDtypeStruct((4 * EP,), jnp.float32),
            jax.ShapeDtypeStruct((2 * NP, 8), jnp.float32),
        ],
        mesh=mesh,
        compiler_params=pltpu.CompilerParams(use_tc_tiling_on_sc=False, needs_layout_passes=False),
        scratch_types=[
            pltpu.VMEM((1024,), jnp.int32),
            pltpu.VMEM((1024,), jnp.int32),
            pltpu.VMEM((1024, 8), jnp.float32),
            pltpu.VMEM((1024, 8), jnp.float32),
            pltpu.VMEM((1024, 8), jnp.float32),
            pltpu.VMEM((4 * 1024,), jnp.float32),
            pltpu.VMEM((16,), jnp.int32),
            pltpu.VMEM((16,), jnp.int32),
            pltpu.VMEM((16,), jnp.int32),
            pltpu.VMEM((16,), jnp.float32),
            pltpu.VMEM((16,), jnp.float32),
            pltpu.VMEM_SHARED((NP, 8), jnp.float32),
        ],
    )
    def pass_a4(src_hbm, dst_hbm, as_hbm, ad_hbm, zeros8_hbm, csv_hbm,
                cdv_hbm, f_out, den_out,
                srcb, dstb, asrow, adrow, fbuf, ftmp, patq, rowq, colq,
                csb, cdb, dsh8):
        scid = lax.axis_index("c")
        sid = lax.axis_index("s")
        row0 = sid * ROWS_PER_TILE
        tile_e0 = (scid * 16 + sid) * (EP // 32)

        pltpu.sync_copy(csv_hbm, csb)
        pltpu.sync_copy(cdv_hbm, cdb)
        cval = (csb[...] + cdb[...])[0]
        lanes = lax.iota(jnp.int32, 16)
        patq[...] = (lanes & 3) * 1024 + (lanes >> 2)
        rowq[...] = lanes >> 2
        colq[...] = lanes & 3

        pltpu.sync_copy(zeros8_hbm.at[pl.ds(0, 1024)], fbuf)
        pltpu.sync_copy(zeros8_hbm, dsh8.at[pl.ds(row0, ROWS_PER_TILE)])
        plsc.subcore_barrier()

        def sg_body(sg, carry):
            eoff = tile_e0 + sg * 1024
            pltpu.sync_copy(src_hbm.at[pl.ds(eoff, 1024)], srcb)
            pltpu.sync_copy(dst_hbm.at[pl.ds(eoff, 1024)], dstb)
            pltpu.sync_copy(as_hbm.at[srcb], asrow)
            pltpu.sync_copy(ad_hbm.at[dstb], adrow)

            def kk_body(kk, carry3):
                ri = rowq[...] + 4 * kk
                ci = colq[...]
                z = (plsc.load_gather(asrow, [ri, ci])
                     + plsc.load_gather(adrow, [ri, ci]))
                f16 = jnp.exp(jnp.maximum(z, 0.2 * z) - cval)
                plsc.store_scatter(fbuf, [ri, ci], f16)
                plsc.store_scatter(ftmp, [patq[...] + 4 * kk], f16)
                return carry3

            lax.fori_loop(0, 256, kk_body, 0)
            pltpu.sync_copy(fbuf, dsh8.at[dstb], add=True)
            for h in range(4):
                pltpu.sync_copy(
                    ftmp.at[pl.ds(h * 1024, 1024)],
                    f_out.at[pl.ds(pl.multiple_of(h * EP + eoff, 8), 1024)])
            return carry

        lax.fori_loop(0, SG_A, sg_body, 0)
        plsc.subcore_barrier()
        pltpu.sync_copy(
            dsh8.at[pl.ds(row0, ROWS_PER_TILE)],
            den_out.at[pl.ds(pl.multiple_of(scid * NP + row0, 8),
                             ROWS_PER_TILE)])

    return pass_a4


def _make_pass_a1():
    mesh = plsc.VectorSubcoreMesh(core_axis_name="c", subcore_axis_name="s")

    @functools.partial(
        pl.kernel,
        out_type=[
            jax.ShapeDtypeStruct((EP,), jnp.float32),
            jax.ShapeDtypeStruct((2 * NP,), jnp.float32),
        ],
        mesh=mesh,
        compiler_params=pltpu.CompilerParams(use_tc_tiling_on_sc=False, needs_layout_passes=False),
        scratch_types=[
            pltpu.VMEM((NP,), jnp.float32),
            pltpu.VMEM((NP,), jnp.float32),
            pltpu.VMEM((1024,), jnp.int32),
            pltpu.VMEM((1024,), jnp.int32),
            pltpu.VMEM((1024,), jnp.float32),
            pltpu.VMEM((16,), jnp.float32),
            pltpu.VMEM((16,), jnp.float32),
            pltpu.VMEM_SHARED((NP,), jnp.float32),
        ],
    )
    def pass_a1(src_hbm, dst_hbm, as_hbm, ad_hbm, zeros1_hbm, csv_hbm,
                cdv_hbm, f_out, den_out,
                as_t, ad_t, srcb, dstb, fb, csb, cdb, dsh):
        scid = lax.axis_index("c")
        sid = lax.axis_index("s")
        row0 = sid * ROWS_PER_TILE
        tile_e0 = (scid * 16 + sid) * (EP // 32)

        pltpu.sync_copy(csv_hbm, csb)
        pltpu.sync_copy(cdv_hbm, cdb)
        cval = (csb[...] + cdb[...])[0]
        pltpu.sync_copy(as_hbm, as_t)
        pltpu.sync_copy(ad_hbm, ad_t)

        pltpu.sync_copy(zeros1_hbm, dsh.at[pl.ds(row0, ROWS_PER_TILE)])
        plsc.subcore_barrier()

        def sg_body(sg, carry):
            eoff = tile_e0 + sg * 1024
            pltpu.sync_copy(src_hbm.at[pl.ds(eoff, 1024)], srcb)
            pltpu.sync_copy(dst_hbm.at[pl.ds(eoff, 1024)], dstb)

            def kk_body(kk, carry3):
                sl = pl.ds(kk * 16, 16)
                z = (plsc.load_gather(as_t, [srcb[sl]])
                     + plsc.load_gather(ad_t, [dstb[sl]]))
                fb[sl] = jnp.exp(jnp.maximum(z, 0.2 * z) - cval)
                return carry3

            lax.fori_loop(0, 64, kk_body, 0)
            pltpu.sync_copy(fb, dsh.at[dstb], add=True)
            pltpu.sync_copy(
                fb, f_out.at[pl.ds(pl.multiple_of(eoff, 8), 1024)])
            return carry

        lax.fori_loop(0, SG_A, sg_body, 0)
        plsc.subcore_barrier()
        pltpu.sync_copy(
            dsh.at[pl.ds(row0, ROWS_PER_TILE)],
            den_out.at[pl.ds(pl.multiple_of(scid * NP + row0, 8),
                             ROWS_PER_TILE)])

    return pass_a1


def _make_sc_layer(heads):
    chunks = 2 * heads
    cps = heads

    mesh = plsc.VectorSubcoreMesh(core_axis_name="c", subcore_axis_name="s")

    @functools.partial(
        pl.kernel,
        out_type=jax.ShapeDtypeStruct((chunks * NP, 32), jnp.float32),
        mesh=mesh,
        compiler_params=pltpu.CompilerParams(use_tc_tiling_on_sc=False, needs_layout_passes=False),
        scratch_types=[
            pltpu.VMEM((1024,), jnp.int32),
            pltpu.VMEM((1024,), jnp.int32),
            pltpu.VMEM((1024,), jnp.float32),
            pltpu.VMEM((512,), jnp.int32),
            pltpu.VMEM((512, 32), jnp.float32),
            pltpu.VMEM_SHARED((NP, 32), jnp.float32),
        ],
    )
    def sc_layer(src_hbm, dst_hbm, f_hbm, hf_hbm, zeros32_hbm,
                 msg_out,
                 srcb, dstb, fsl, hidx, rows, acc):
        scid = lax.axis_index("c")
        sid = lax.axis_index("s")
        row0 = sid * ROWS_PER_TILE
        tile_e0 = sid * E_PER_TILE

        for c in range(cps):
            cg = scid * cps + c
            head = cg // 2
            goff = cg * NP

            pltpu.sync_copy(zeros32_hbm, acc.at[pl.ds(row0, ROWS_PER_TILE)])
            plsc.subcore_barrier()

            def sg_body(sg, carry):
                eoff = tile_e0 + sg * 1024
                pltpu.sync_copy(src_hbm.at[pl.ds(eoff, 1024)], srcb)
                pltpu.sync_copy(dst_hbm.at[pl.ds(eoff, 1024)], dstb)
                pltpu.sync_copy(
                    f_hbm.at[pl.ds(pl.multiple_of(head * EP + eoff, 8), 1024)],
                    fsl)

                for hh in range(2):
                    h0 = hh * 512

                    def idx_body(k, carry2):
                        hidx[pl.ds(k * 16, 16)] = (
                            srcb[pl.ds(h0 + k * 16, 16)] + goff)
                        return carry2

                    lax.fori_loop(0, 32, idx_body, 0)
                    pltpu.sync_copy(hf_hbm.at[hidx], rows)

                    def mul_body(kk, carry3):
                        f16 = fsl[pl.ds(h0 + kk * 16, 16)]
                        for j in range(16):
                            e = kk * 16 + j
                            fs = f16[j]
                            rows[e, pl.ds(0, 16)] = rows[e, pl.ds(0, 16)] * fs
                            rows[e, pl.ds(16, 16)] = (
                                rows[e, pl.ds(16, 16)] * fs)
                        return carry3

                    lax.fori_loop(0, 32, mul_body, 0)
                    pltpu.sync_copy(rows, acc.at[dstb.at[pl.ds(h0, 512)]],
                                    add=True)
                return carry

            lax.fori_loop(0, SG_PER_TILE, sg_body, 0)
            plsc.subcore_barrier()

            pltpu.sync_copy(
                acc.at[pl.ds(row0, ROWS_PER_TILE)],
                msg_out.at[pl.ds(pl.multiple_of(goff + row0, 8),
                                 ROWS_PER_TILE)])
            plsc.subcore_barrier()

    return sc_layer


_pass_l1 = _make_pass_a4()
_pass_l2 = _make_pass_a1()
_sc_layer1 = _make_sc_layer(4)
_sc_layer2 = _make_sc_layer(1)


def _att_matrix(att, heads, dim):
    m = jnp.zeros((heads * dim, 8), jnp.float32)
    for h in range(heads):
        m = m.at[h * dim:(h + 1) * dim, h].set(att[0, h, :])
    return m


def kernel(x, edge_index, W1, att_src1, att_dst1, b1, W2, att_src2, att_dst2,
           b2, Wc, bc):
    f32 = jnp.float32

    x8 = jnp.zeros((NP, 8), f32).at[:N, :F_IN].set(x)
    w1p = jnp.zeros((8, 4 * HID), f32).at[:F_IN].set(W1)
    ms1 = _att_matrix(att_src1, 4, HID)
    md1 = _att_matrix(att_dst1, 4, HID)
    ms2 = _att_matrix(att_src2, 1, HID)
    md2 = _att_matrix(att_dst2, 1, HID)

    loops = jnp.arange(N, dtype=jnp.int32)
    pad = jnp.full((EP - E_LOOP,), NP - 1, jnp.int32)
    src = jnp.concatenate([edge_index[0].astype(jnp.int32), loops, pad])
    dst = jnp.concatenate([edge_index[1].astype(jnp.int32), loops, pad])

    zeros32 = jnp.zeros((ROWS_PER_TILE, 32), f32)
    zeros8 = jnp.zeros((ROWS_PER_TILE, 8), f32)
    zeros1 = jnp.zeros((ROWS_PER_TILE,), f32)

    h1, as1, ad1, cs1, cd1 = _dense1(x8, w1p, ms1, md1)
    hf1 = h1.reshape(NP, 8, 32).transpose(1, 0, 2).reshape(8 * NP, 32)
    f1, den1 = _pass_l1(src, dst, as1, ad1, zeros8,
                        cs1.reshape(16), cd1.reshape(16))
    msg1 = _sc_layer1(src, dst, f1, hf1, zeros32)
    msg1r = msg1.reshape(8, NP, 32).transpose(1, 0, 2).reshape(NP, 4 * HID)

    b1r = b1.reshape(1, 4 * HID)
    r1 = jnp.repeat(jnp.eye(4, dtype=f32), HID, axis=1)
    h2, as2, ad2, cs2, cd2 = _dense2(msg1r, den1[:NP], den1[NP:], r1, b1r,
                                     W2, ms2, md2)
    hf2 = h2.reshape(NP, 2, 32).transpose(1, 0, 2).reshape(2 * NP, 32)
    f2, den2 = _pass_l2(src, dst, as2[:, 0], ad2[:, 0], zeros1,
                        cs2.reshape(16), cd2.reshape(16))
    msg2 = _sc_layer2(src, dst, f2, hf2, zeros32)
    msg2r = msg2.reshape(2, NP, 32).transpose(1, 0, 2).reshape(NP, HID)

    b2r = b2.reshape(1, HID)
    wcp = jnp.zeros((HID, 128), f32).at[:, 0:NUM_CLASSES].set(Wc)
    bcp = jnp.zeros((1, 128), f32).at[0, 0:NUM_CLASSES].set(bc)
    out = _dense3(msg2r, den2[:NP].reshape(NP, 1), den2[NP:].reshape(NP, 1),
                  b2r, wcp, bcp)
    return out[:N]


# speedup vs baseline: 41.3065x; 1.0467x over previous
import functools

import jax
import jax.numpy as jnp
from jax import lax
from jax.experimental import pallas as pl
from jax.experimental.pallas import tpu as pltpu
from jax.experimental.pallas import tpu_sc as plsc

N = 50000
F_IN = 6
HID = 64
NUM_CLASSES = 2

NP = 50176
ROWS_PER_TILE = NP // 16
E_RAW = 800000
E_LOOP = E_RAW + N
EP = 851968
E_PER_TILE = EP // 16
SG_PER_TILE = E_PER_TILE // 1024

NB = 512
GRID = NP // NB


def _k1_body(x_ref, w1_ref, ms_ref, md_ref, h_ref, as_ref, ad_ref, cs_ref, cd_ref):
    i = pl.program_id(0)
    h = jnp.dot(x_ref[...], w1_ref[...], preferred_element_type=jnp.float32)
    a_s = jnp.dot(h, ms_ref[...], preferred_element_type=jnp.float32)
    a_d = jnp.dot(h, md_ref[...], preferred_element_type=jnp.float32)
    for hh in range(8):
        h_ref[hh] = h[:, 32 * hh:32 * (hh + 1)]
    as_ref[...] = a_s
    ad_ref[...] = a_d
    ms_blk = jnp.max(a_s[:, 0:4])
    md_blk = jnp.max(a_d[:, 0:4])

    @pl.when(i == 0)
    def _():
        cs_ref[...] = jnp.full((1, 16), -1e30, jnp.float32)
        cd_ref[...] = jnp.full((1, 16), -1e30, jnp.float32)

    cs_ref[...] = jnp.maximum(cs_ref[...], ms_blk)
    cd_ref[...] = jnp.maximum(cd_ref[...], md_blk)


def _dense1(x8, w1p, ms1, md1):
    return pl.pallas_call(
        _k1_body,
        grid=(GRID,),
        in_specs=[
            pl.BlockSpec((NB, 8), lambda i: (i, 0)),
            pl.BlockSpec((8, 4 * HID), lambda i: (0, 0)),
            pl.BlockSpec((4 * HID, 8), lambda i: (0, 0)),
            pl.BlockSpec((4 * HID, 8), lambda i: (0, 0)),
        ],
        out_specs=[
            pl.BlockSpec((8, NB, 32), lambda i: (0, i, 0)),
            pl.BlockSpec((NB, 8), lambda i: (i, 0)),
            pl.BlockSpec((NB, 8), lambda i: (i, 0)),
            pl.BlockSpec((1, 16), lambda i: (0, 0)),
            pl.BlockSpec((1, 16), lambda i: (0, 0)),
        ],
        out_shape=[
            jax.ShapeDtypeStruct((8, NP, 32), jnp.float32),
            jax.ShapeDtypeStruct((NP, 8), jnp.float32),
            jax.ShapeDtypeStruct((NP, 8), jnp.float32),
            jax.ShapeDtypeStruct((1, 16), jnp.float32),
            jax.ShapeDtypeStruct((1, 16), jnp.float32),
        ],
    )(x8, w1p, ms1, md1)


def _k3_body(msg_ref, da_ref, db_ref, r1_ref, b1_ref, w2_ref, ms_ref, md_ref,
             h2_ref, as_ref, ad_ref, cs_ref, cd_ref):
    i = pl.program_id(0)
    d4 = (da_ref[...] + db_ref[...])[:, 0:4] + 1e-16
    dexp = jnp.dot(d4, r1_ref[...], preferred_element_type=jnp.float32)
    msg = jnp.concatenate([msg_ref[hh] for hh in range(8)], axis=1)
    out1 = jnp.maximum(msg / dexp + b1_ref[...], 0.0)
    h2 = jnp.dot(out1, w2_ref[...], preferred_element_type=jnp.float32)
    a_s = jnp.dot(h2, ms_ref[...], preferred_element_type=jnp.float32)
    a_d = jnp.dot(h2, md_ref[...], preferred_element_type=jnp.float32)
    h2_ref[0] = h2[:, 0:32]
    h2_ref[1] = h2[:, 32:64]
    as_ref[...] = a_s
    ad_ref[...] = a_d
    ms_blk = jnp.max(a_s[:, 0:1])
    md_blk = jnp.max(a_d[:, 0:1])

    @pl.when(i == 0)
    def _():
        cs_ref[...] = jnp.full((1, 16), -1e30, jnp.float32)
        cd_ref[...] = jnp.full((1, 16), -1e30, jnp.float32)

    cs_ref[...] = jnp.maximum(cs_ref[...], ms_blk)
    cd_ref[...] = jnp.maximum(cd_ref[...], md_blk)


def _dense2(msg1, da, db, r1, b1r, w2, ms2, md2):
    return pl.pallas_call(
        _k3_body,
        grid=(GRID,),
        in_specs=[
            pl.BlockSpec((8, NB, 32), lambda i: (0, i, 0)),
            pl.BlockSpec((NB, 8), lambda i: (i, 0)),
            pl.BlockSpec((NB, 8), lambda i: (i, 0)),
            pl.BlockSpec((4, 4 * HID), lambda i: (0, 0)),
            pl.BlockSpec((1, 4 * HID), lambda i: (0, 0)),
            pl.BlockSpec((4 * HID, HID), lambda i: (0, 0)),
            pl.BlockSpec((HID, 8), lambda i: (0, 0)),
            pl.BlockSpec((HID, 8), lambda i: (0, 0)),
        ],
        out_specs=[
            pl.BlockSpec((2, NB, 32), lambda i: (0, i, 0)),
            pl.BlockSpec((NB, 8), lambda i: (i, 0)),
            pl.BlockSpec((NB, 8), lambda i: (i, 0)),
            pl.BlockSpec((1, 16), lambda i: (0, 0)),
            pl.BlockSpec((1, 16), lambda i: (0, 0)),
        ],
        out_shape=[
            jax.ShapeDtypeStruct((2, NP, 32), jnp.float32),
            jax.ShapeDtypeStruct((NP, 8), jnp.float32),
            jax.ShapeDtypeStruct((NP, 8), jnp.float32),
            jax.ShapeDtypeStruct((1, 16), jnp.float32),
            jax.ShapeDtypeStruct((1, 16), jnp.float32),
        ],
    )(msg1, da, db, r1, b1r, w2, ms2, md2)


def _k6_body(msg_ref, da_ref, db_ref, b2_ref, wc_ref, bc_ref, out_ref):
    d = (da_ref[...] + db_ref[...]) + 1e-16
    msg = jnp.concatenate([msg_ref[0], msg_ref[1]], axis=1)
    out2 = jnp.maximum(msg / d + b2_ref[...], 0.0)
    lg = jnp.dot(out2, wc_ref[...], preferred_element_type=jnp.float32) + bc_ref[...]
    l0 = lg[:, 0:1]
    l1 = lg[:, 1:2]
    m = jnp.maximum(l0, l1)
    s = m + jnp.log(jnp.exp(l0 - m) + jnp.exp(l1 - m))
    out_ref[...] = lg[:, 0:2] - s


def _dense3(msg2, da, db, b2r, wcp, bcp):
    return pl.pallas_call(
        _k6_body,
        grid=(GRID,),
        in_specs=[
            pl.BlockSpec((2, NB, 32), lambda i: (0, i, 0)),
            pl.BlockSpec((NB, 1), lambda i: (i, 0)),
            pl.BlockSpec((NB, 1), lambda i: (i, 0)),
            pl.BlockSpec((1, HID), lambda i: (0, 0)),
            pl.BlockSpec((HID, 128), lambda i: (0, 0)),
            pl.BlockSpec((1, 128), lambda i: (0, 0)),
        ],
        out_specs=pl.BlockSpec((NB, 2), lambda i: (i, 0)),
        out_shape=jax.ShapeDtypeStruct((NP, 2), jnp.float32),
    )(msg2, da, db, b2r, wcp, bcp)


SG_A = EP // 32 // 1024


def _make_pass_a4():
    mesh = plsc.VectorSubcoreMesh(core_axis_name="c", subcore_axis_name="s")

    @functools.partial(
        pl.kernel,
        out_type=[
            jax.ShapeDtypeStruct((4 * EP,), jnp.float32),
            jax.ShapeDtypeStruct((2 * NP, 8), jnp.float32),
        ],
        mesh=mesh,
        compiler_params=pltpu.CompilerParams(use_tc_tiling_on_sc=False, needs_layout_passes=False),
        scratch_types=[
            pltpu.VMEM((1024,), jnp.int32),
            pltpu.VMEM((1024,), jnp.int32),
            pltpu.VMEM((1024, 8), jnp.float32),
            pltpu.VMEM((1024, 8), jnp.float32),
            pltpu.VMEM((1024, 8), jnp.float32),
            pltpu.VMEM((4 * 1024,), jnp.float32),
            pltpu.VMEM((16,), jnp.int32),
            pltpu.VMEM((16,), jnp.int32),
            pltpu.VMEM((16,), jnp.int32),
            pltpu.VMEM((16,), jnp.float32),
            pltpu.VMEM((16,), jnp.float32),
            pltpu.VMEM_SHARED((NP, 8), jnp.float32),
        ],
    )
    def pass_a4(src_hbm, dst_hbm, as_hbm, ad_hbm, zeros8_hbm, csv_hbm,
                cdv_hbm, f_out, den_out,
                srcb, dstb, asrow, adrow, fbuf, ftmp, patq, rowq, colq,
                csb, cdb, dsh8):
        scid = lax.axis_index("c")
        sid = lax.axis_index("s")
        row0 = sid * ROWS_PER_TILE
        tile_e0 = (scid * 16 + sid) * (EP // 32)

        pltpu.sync_copy(csv_hbm, csb)
        pltpu.sync_copy(cdv_hbm, cdb)
        cval = (csb[...] + cdb[...])[0]
        lanes = lax.iota(jnp.int32, 16)
        patq[...] = (lanes & 3) * 1024 + (lanes >> 2)
        rowq[...] = lanes >> 2
        colq[...] = lanes & 3

        pltpu.sync_copy(zeros8_hbm.at[pl.ds(0, 1024)], fbuf)
        pltpu.sync_copy(zeros8_hbm, dsh8.at[pl.ds(row0, ROWS_PER_TILE)])
        plsc.subcore_barrier()

        def sg_body(sg, carry):
            eoff = tile_e0 + sg * 1024
            pltpu.sync_copy(src_hbm.at[pl.ds(eoff, 1024)], srcb)
            pltpu.sync_copy(dst_hbm.at[pl.ds(eoff, 1024)], dstb)
            pltpu.sync_copy(as_hbm.at[srcb], asrow)
            pltpu.sync_copy(ad_hbm.at[dstb], adrow)

            def kk_body(kk, carry3):
                ri = rowq[...] + 4 * kk
                ci = colq[...]
                z = (plsc.load_gather(asrow, [ri, ci])
                     + plsc.load_gather(adrow, [ri, ci]))
                f16 = jnp.exp(jnp.maximum(z, 0.2 * z) - cval)
                plsc.store_scatter(fbuf, [ri, ci], f16)
                plsc.store_scatter(ftmp, [patq[...] + 4 * kk], f16)
                return carry3

            lax.fori_loop(0, 256, kk_body, 0)
            pltpu.sync_copy(fbuf, dsh8.at[dstb], add=True)
            for h in range(4):
                pltpu.sync_copy(
                    ftmp.at[pl.ds(h * 1024, 1024)],
                    f_out.at[pl.ds(pl.multiple_of(h * EP + eoff, 8), 1024)])
            return carry

        lax.fori_loop(0, SG_A, sg_body, 0)
        plsc.subcore_barrier()
        pltpu.sync_copy(
            dsh8.at[pl.ds(row0, ROWS_PER_TILE)],
            den_out.at[pl.ds(pl.multiple_of(scid * NP + row0, 8),
                             ROWS_PER_TILE)])

    return pass_a4


def _make_pass_a1():
    mesh = plsc.VectorSubcoreMesh(core_axis_name="c", subcore_axis_name="s")

    @functools.partial(
        pl.kernel,
        out_type=[
            jax.ShapeDtypeStruct((EP,), jnp.float32),
            jax.ShapeDtypeStruct((2 * NP,), jnp.float32),
        ],
        mesh=mesh,
        compiler_params=pltpu.CompilerParams(use_tc_tiling_on_sc=False, needs_layout_passes=False),
        scratch_types=[
            pltpu.VMEM((NP,), jnp.float32),
            pltpu.VMEM((NP,), jnp.float32),
            pltpu.VMEM((1024,), jnp.int32),
            pltpu.VMEM((1024,), jnp.int32),
            pltpu.VMEM((1024,), jnp.float32),
            pltpu.VMEM((16,), jnp.float32),
            pltpu.VMEM((16,), jnp.float32),
            pltpu.VMEM_SHARED((NP,), jnp.float32),
        ],
    )
    def pass_a1(src_hbm, dst_hbm, as_hbm, ad_hbm, zeros1_hbm, csv_hbm,
                cdv_hbm, f_out, den_out,
                as_t, ad_t, srcb, dstb, fb, csb, cdb, dsh):
        scid = lax.axis_index("c")
        sid = lax.axis_index("s")
        row0 = sid * ROWS_PER_TILE
        tile_e0 = (scid * 16 + sid) * (EP // 32)

        pltpu.sync_copy(csv_hbm, csb)
        pltpu.sync_copy(cdv_hbm, cdb)
        cval = (csb[...] + cdb[...])[0]
        pltpu.sync_copy(as_hbm, as_t)
        pltpu.sync_copy(ad_hbm, ad_t)

        pltpu.sync_copy(zeros1_hbm, dsh.at[pl.ds(row0, ROWS_PER_TILE)])
        plsc.subcore_barrier()

        def sg_body(sg, carry):
            eoff = tile_e0 + sg * 1024
            pltpu.sync_copy(src_hbm.at[pl.ds(eoff, 1024)], srcb)
            pltpu.sync_copy(dst_hbm.at[pl.ds(eoff, 1024)], dstb)

            def kk_body(kk, carry3):
                sl = pl.ds(kk * 16, 16)
                z = (plsc.load_gather(as_t, [srcb[sl]])
                     + plsc.load_gather(ad_t, [dstb[sl]]))
                fb[sl] = jnp.exp(jnp.maximum(z, 0.2 * z) - cval)
                return carry3

            lax.fori_loop(0, 64, kk_body, 0)
            pltpu.sync_copy(fb, dsh.at[dstb], add=True)
            pltpu.sync_copy(
                fb, f_out.at[pl.ds(pl.multiple_of(eoff, 8), 1024)])
            return carry

        lax.fori_loop(0, SG_A, sg_body, 0)
        plsc.subcore_barrier()
        pltpu.sync_copy(
            dsh.at[pl.ds(row0, ROWS_PER_TILE)],
            den_out.at[pl.ds(pl.multiple_of(scid * NP + row0, 8),
                             ROWS_PER_TILE)])

    return pass_a1


def _make_sc_layer(heads):
    chunks = 2 * heads
    cps = heads

    mesh = plsc.VectorSubcoreMesh(core_axis_name="c", subcore_axis_name="s")

    @functools.partial(
        pl.kernel,
        out_type=jax.ShapeDtypeStruct((chunks * NP, 32), jnp.float32),
        mesh=mesh,
        compiler_params=pltpu.CompilerParams(use_tc_tiling_on_sc=False, needs_layout_passes=False),
        scratch_types=[
            pltpu.VMEM((1024,), jnp.int32),
            pltpu.VMEM((1024,), jnp.int32),
            pltpu.VMEM((1024,), jnp.float32),
            pltpu.VMEM((512,), jnp.int32),
            pltpu.VMEM((512, 32), jnp.float32),
            pltpu.VMEM_SHARED((NP, 32), jnp.float32),
        ],
    )
    def sc_layer(src_hbm, dst_hbm, f_hbm, hf_hbm, zeros32_hbm,
                 msg_out,
                 srcb, dstb, fsl, hidx, rows, acc):
        scid = lax.axis_index("c")
        sid = lax.axis_index("s")
        row0 = sid * ROWS_PER_TILE
        tile_e0 = sid * E_PER_TILE

        for c in range(cps):
            cg = scid * cps + c
            head = cg // 2
            goff = cg * NP

            pltpu.sync_copy(zeros32_hbm, acc.at[pl.ds(row0, ROWS_PER_TILE)])
            plsc.subcore_barrier()

            def sg_body(sg, carry):
                eoff = tile_e0 + sg * 1024
                pltpu.sync_copy(src_hbm.at[pl.ds(eoff, 1024)], srcb)
                pltpu.sync_copy(dst_hbm.at[pl.ds(eoff, 1024)], dstb)
                pltpu.sync_copy(
                    f_hbm.at[pl.ds(pl.multiple_of(head * EP + eoff, 8), 1024)],
                    fsl)

                for hh in range(2):
                    h0 = hh * 512

                    def idx_body(k, carry2):
                        hidx[pl.ds(k * 16, 16)] = (
                            srcb[pl.ds(h0 + k * 16, 16)] + goff)
                        return carry2

                    lax.fori_loop(0, 32, idx_body, 0)
                    pltpu.sync_copy(hf_hbm.at[hidx], rows)

                    def mul_body(kk, carry3):
                        f16 = fsl[pl.ds(h0 + kk * 16, 16)]
                        for j in range(16):
                            e = kk * 16 + j
                            fs = f16[j]
                            rows[e, pl.ds(0, 16)] = rows[e, pl.ds(0, 16)] * fs
                            rows[e, pl.ds(16, 16)] = (
                                rows[e, pl.ds(16, 16)] * fs)
                        return carry3

                    lax.fori_loop(0, 32, mul_body, 0)
                    pltpu.sync_copy(rows, acc.at[dstb.at[pl.ds(h0, 512)]],
                                    add=True)
                return carry

            lax.fori_loop(0, SG_PER_TILE, sg_body, 0)
            plsc.subcore_barrier()

            pltpu.sync_copy(
                acc.at[pl.ds(row0, ROWS_PER_TILE)],
                msg_out.at[pl.ds(pl.multiple_of(goff + row0, 8),
                                 ROWS_PER_TILE)])
            plsc.subcore_barrier()

    return sc_layer


_pass_l1 = _make_pass_a4()
_pass_l2 = _make_pass_a1()
_sc_layer1 = _make_sc_layer(4)
_sc_layer2 = _make_sc_layer(1)


def _att_matrix(att, heads, dim):
    m = jnp.zeros((heads * dim, 8), jnp.float32)
    for h in range(heads):
        m = m.at[h * dim:(h + 1) * dim, h].set(att[0, h, :])
    return m


def kernel(x, edge_index, W1, att_src1, att_dst1, b1, W2, att_src2, att_dst2,
           b2, Wc, bc):
    f32 = jnp.float32

    x8 = jnp.zeros((NP, 8), f32).at[:N, :F_IN].set(x)
    w1p = jnp.zeros((8, 4 * HID), f32).at[:F_IN].set(W1)
    ms1 = _att_matrix(att_src1, 4, HID)
    md1 = _att_matrix(att_dst1, 4, HID)
    ms2 = _att_matrix(att_src2, 1, HID)
    md2 = _att_matrix(att_dst2, 1, HID)

    loops = jnp.arange(N, dtype=jnp.int32)
    pad = jnp.full((EP - E_LOOP,), NP - 1, jnp.int32)
    src = jnp.concatenate([edge_index[0].astype(jnp.int32), loops, pad])
    dst = jnp.concatenate([edge_index[1].astype(jnp.int32), loops, pad])

    zeros32 = jnp.zeros((ROWS_PER_TILE, 32), f32)
    zeros8 = jnp.zeros((ROWS_PER_TILE, 8), f32)
    zeros1 = jnp.zeros((ROWS_PER_TILE,), f32)

    h1, as1, ad1, cs1, cd1 = _dense1(x8, w1p, ms1, md1)
    hf1 = h1.reshape(8 * NP, 32)
    f1, den1 = _pass_l1(src, dst, as1, ad1, zeros8,
                        cs1.reshape(16), cd1.reshape(16))
    msg1 = _sc_layer1(src, dst, f1, hf1, zeros32)

    b1r = b1.reshape(1, 4 * HID)
    r1 = jnp.repeat(jnp.eye(4, dtype=f32), HID, axis=1)
    h2, as2, ad2, cs2, cd2 = _dense2(msg1.reshape(8, NP, 32), den1[:NP],
                                     den1[NP:], r1, b1r, W2, ms2, md2)
    hf2 = h2.reshape(2 * NP, 32)
    f2, den2 = _pass_l2(src, dst, as2[:, 0], ad2[:, 0], zeros1,
                        cs2.reshape(16), cd2.reshape(16))
    msg2 = _sc_layer2(src, dst, f2, hf2, zeros32)

    b2r = b2.reshape(1, HID)
    wcp = jnp.zeros((HID, 128), f32).at[:, 0:NUM_CLASSES].set(Wc)
    bcp = jnp.zeros((1, 128), f32).at[0, 0:NUM_CLASSES].set(bc)
    out = _dense3(msg2.reshape(2, NP, 32), den2[:NP].reshape(NP, 1),
                  den2[NP:].reshape(NP, 1), b2r, wcp, bcp)
    return out[:N]
